# baseline (speedup 1.0000x reference)
import dataclasses
import functools

import jax
import jax.numpy as jnp
from jax import lax
from jax.experimental import pallas as pl
from jax.experimental.pallas import tpu as pltpu
from jax.experimental.pallas import tpu_sc as plsc

_NC = 2
_NS = 16
_L = 16
_NW = _NC * _NS

_COPY_BLOCK = 10000


def _copy_body(x_ref, o_ref):
    o_ref[...] = x_ref[...]


def _tc_copy(bank):
    n, d = bank.shape
    blk = _COPY_BLOCK
    assert n % blk == 0
    return pl.pallas_call(
        _copy_body,
        grid=(n // blk,),
        in_specs=[pl.BlockSpec((blk, d), lambda i: (i, 0))],
        out_specs=pl.BlockSpec((blk, d), lambda i: (i, 0)),
        out_shape=jax.ShapeDtypeStruct((n, d), bank.dtype),
    )(bank)


@functools.cache
def _make_sc_scatter(n_rows, d, b):
    w = b // _NW
    assert b % (_NW * _L) == 0

    mesh = plsc.VectorSubcoreMesh(core_axis_name="c", subcore_axis_name="s")
    cp = pltpu.CompilerParams()
    fields = pltpu.CompilerParams.__dataclass_fields__
    if "needs_layout_passes" in fields:
        cp = dataclasses.replace(cp, needs_layout_passes=False)
    if "use_tc_tiling_on_sc" in fields:
        cp = dataclasses.replace(cp, use_tc_tiling_on_sc=False)

    @functools.partial(
        pl.kernel,
        out_type=(),
        mesh=mesh,
        compiler_params=cp,
        scratch_types=[
            pltpu.VMEM((b,), jnp.int32),
            pltpu.VMEM((n_rows,), jnp.int32),
            pltpu.VMEM((w,), jnp.int32),
            pltpu.VMEM((w,), jnp.int32),
            pltpu.VMEM((w, d), jnp.float32),
            pltpu.SemaphoreType.DMA,
        ],
    )
    def sc_scatter(val_hbm, idx_hbm, out_hbm,
                   idx_buf, pos_table, win_buf, idx_win, rows_v, sem):
        iota = lax.iota(jnp.int32, _L)
        wid = lax.axis_index("s") * _NC + lax.axis_index("c")
        base = wid * w

        pltpu.sync_copy(idx_hbm, idx_buf)

        @pl.loop(0, b, step=_L)
        def _(i):
            v = idx_buf[pl.ds(i, _L)]
            plsc.store_scatter(pos_table, [v], iota + i)

        def w_cond(changed):
            return changed > 0

        def w_body(_):
            @pl.loop(0, b, step=_L, init_carry=jnp.int32(0))
            def changed(i, ch):
                v = idx_buf[pl.ds(i, _L)]
                pos = iota + i
                t = plsc.load_gather(pos_table, [v])
                m = pos > t
                plsc.store_scatter(pos_table, [v], pos, mask=m)
                return ch + jnp.sum(m.astype(jnp.int32))

            return changed

        lax.while_loop(w_cond, w_body, jnp.int32(1))

        @pl.loop(0, w, step=_L)
        def _(r):
            v = idx_buf[pl.ds(base + r, _L)]
            win_buf[pl.ds(r, _L)] = plsc.load_gather(pos_table, [v])

        pltpu.sync_copy(idx_hbm.at[pl.ds(base, w)], idx_win)
        pltpu.async_copy(val_hbm.at[win_buf], rows_v, sem).wait()
        pltpu.async_copy(rows_v, out_hbm.at[idx_win], sem).wait()

    return sc_scatter


def kernel(bank, idx, val):
    n, d = bank.shape
    out_ref = jax.new_ref(_tc_copy(bank))
    _make_sc_scatter(n, d, idx.shape[0])(val, idx, out_ref)
    return out_ref[...]


# speedup vs baseline: 1.2965x; 1.2965x over previous
import dataclasses
import functools

import jax
import jax.numpy as jnp
from jax import lax
from jax.experimental import pallas as pl
from jax.experimental.pallas import tpu as pltpu
from jax.experimental.pallas import tpu_sc as plsc

_NC = 2
_NS = 16
_L = 16
_NW = _NC * _NS

_COPY_BLOCK = 10000


def _copy_body(x_ref, o_ref):
    o_ref[...] = x_ref[...]


def _tc_copy(bank):
    n, d = bank.shape
    blk = _COPY_BLOCK
    assert n % blk == 0
    return pl.pallas_call(
        _copy_body,
        grid=(n // blk,),
        in_specs=[pl.BlockSpec((blk, d), lambda i: (i, 0))],
        out_specs=pl.BlockSpec((blk, d), lambda i: (i, 0)),
        out_shape=jax.ShapeDtypeStruct((n, d), bank.dtype),
    )(bank)


@functools.cache
def _make_sc_scatter(n_rows, d, b):
    w = b // _NW
    assert b % (_NW * _L) == 0

    mesh = plsc.VectorSubcoreMesh(core_axis_name="c", subcore_axis_name="s")
    cp = pltpu.CompilerParams()
    fields = pltpu.CompilerParams.__dataclass_fields__
    if "needs_layout_passes" in fields:
        cp = dataclasses.replace(cp, needs_layout_passes=False)
    if "use_tc_tiling_on_sc" in fields:
        cp = dataclasses.replace(cp, use_tc_tiling_on_sc=False)

    @functools.partial(
        pl.kernel,
        out_type=(),
        mesh=mesh,
        compiler_params=cp,
        scratch_types=[
            pltpu.VMEM((b,), jnp.int32),
            pltpu.VMEM((n_rows,), jnp.int32),
            pltpu.VMEM((w,), jnp.int32),
            pltpu.VMEM((w,), jnp.int32),
            pltpu.VMEM((w, d), jnp.float32),
            pltpu.SemaphoreType.DMA,
        ],
    )
    def sc_scatter(val_hbm, idx_hbm, out_hbm,
                   idx_buf, pos_table, win_buf, idx_win, rows_v, sem):
        iota = lax.iota(jnp.int32, _L)
        wid = lax.axis_index("s") * _NC + lax.axis_index("c")
        base = wid * w

        pltpu.sync_copy(idx_hbm, idx_buf)

        @pl.loop(0, b, step=_L)
        def _(i):
            v = idx_buf[pl.ds(i, _L)]
            plsc.store_scatter(pos_table, [v], iota + i)

        def w_cond(changed):
            return changed > 0

        def w_body(_):
            @pl.loop(0, b, step=_L, init_carry=jnp.int32(0))
            def changed(i, ch):
                v = idx_buf[pl.ds(i, _L)]
                pos = iota + i
                t = plsc.load_gather(pos_table, [v])
                m = pos > t
                plsc.store_scatter(pos_table, [v], pos, mask=m)
                return ch + jnp.sum(m.astype(jnp.int32))

            return changed

        lax.while_loop(w_cond, w_body, jnp.int32(1))

        @pl.loop(0, w, step=_L)
        def _(r):
            v = idx_buf[pl.ds(base + r, _L)]
            win_buf[pl.ds(r, _L)] = plsc.load_gather(pos_table, [v])

        pltpu.sync_copy(idx_hbm.at[pl.ds(base, w)], idx_win)
        pltpu.async_copy(val_hbm.at[win_buf], rows_v, sem).wait()
        pltpu.async_copy(rows_v, out_hbm.at[idx_win], sem).wait()

    return sc_scatter


def kernel(bank, idx, val):
    n, d = bank.shape
    out_ref = jax.new_ref(bank)
    _make_sc_scatter(n, d, idx.shape[0])(val, idx, out_ref)
    return out_ref[...]
